# baseline (speedup 1.0000x reference)
import functools

import jax
import jax.numpy as jnp
from jax import lax
from jax.experimental import pallas as pl
from jax.experimental.pallas import tpu as pltpu
from jax.experimental.pallas import tpu_sc as plsc

_B, _S, _D = 4, 2048, 128
_NC, _NS = 2, 16
_NW = _NC * _NS
_PCH = _S // _NW
_L = 16

_mesh = plsc.VectorSubcoreMesh(core_axis_name="c", subcore_axis_name="s")


@functools.partial(
    pl.kernel,
    mesh=_mesh,
    out_type=jax.ShapeDtypeStruct((_B, _S, _D), jnp.float32),
    scratch_types=[
        pltpu.VMEM((_B, _PCH), jnp.int32),
        pltpu.VMEM((_PCH, _D), jnp.float32),
        pltpu.VMEM((_B, _PCH, _D), jnp.float32),
        pltpu.SemaphoreType.DMA,
        pltpu.SemaphoreType.DMA,
        pltpu.SemaphoreType.DMA,
        pltpu.SemaphoreType.DMA,
        pltpu.SemaphoreType.DMA,
        pltpu.SemaphoreType.DMA,
        pltpu.SemaphoreType.DMA,
    ],
)
def _emb(x_hbm, table_hbm, pos_hbm, out_hbm,
         idx_v, pos_v, rows_v, isem, psem, gs0, gs1, gs2, gs3, ws):
    wid = lax.axis_index("s") * _NC + lax.axis_index("c")
    s0 = wid * _PCH
    gsems = (gs0, gs1, gs2, gs3)

    icps = [
        pltpu.async_copy(x_hbm.at[b, pl.ds(s0, _PCH)], idx_v.at[b], isem)
        for b in range(_B)
    ]
    pcp = pltpu.async_copy(pos_hbm.at[pl.ds(s0, _PCH)], pos_v, psem)
    for cp in icps:
        cp.wait()

    gcps = [
        pltpu.async_copy(table_hbm.at[idx_v.at[b]], rows_v.at[b], gsems[b])
        for b in range(_B)
    ]
    pcp.wait()

    wcps = []
    for b in range(_B):
        gcps[b].wait()

        def add_rows(r2, carry, _b=b):
            for rr in range(2):
                r = r2 * 2 + rr
                for j in range(_D // _L):
                    sl = pl.ds(j * _L, _L)
                    rows_v[_b, r, sl] = rows_v[_b, r, sl] + pos_v[r, sl]
            return carry

        lax.fori_loop(0, _PCH // 2, add_rows, 0)
        wcps.append(pltpu.async_copy(
            rows_v.at[b], out_hbm.at[b, pl.ds(s0, _PCH)], ws))
    for cp in wcps:
        cp.wait()


def kernel(x, table, pos_table):
    return _emb(x, table, pos_table)


# speedup vs baseline: 1.0199x; 1.0199x over previous
import functools

import jax
import jax.numpy as jnp
from jax import lax
from jax.experimental import pallas as pl
from jax.experimental.pallas import tpu as pltpu
from jax.experimental.pallas import tpu_sc as plsc

_B, _S, _D = 4, 2048, 128
_NC, _NS = 2, 16
_NW = _NC * _NS
_PCH = _S // _NW
_L = 16

_mesh = plsc.VectorSubcoreMesh(core_axis_name="c", subcore_axis_name="s")


@functools.partial(
    pl.kernel,
    mesh=_mesh,
    out_type=jax.ShapeDtypeStruct((_B, _S, _D), jnp.float32),
    scratch_types=[
        pltpu.VMEM((_B, _PCH), jnp.int32),
        pltpu.VMEM((_PCH, _D), jnp.float32),
        pltpu.VMEM((_B, _PCH, _D), jnp.float32),
        pltpu.SemaphoreType.DMA,
        pltpu.SemaphoreType.DMA,
        pltpu.SemaphoreType.DMA,
        pltpu.SemaphoreType.DMA,
        pltpu.SemaphoreType.DMA,
        pltpu.SemaphoreType.DMA,
        pltpu.SemaphoreType.DMA,
    ],
)
def _emb(x_hbm, table_hbm, pos_hbm, out_hbm,
         idx_v, pos_v, rows_v, isem, psem, gs0, gs1, gs2, gs3, ws):
    wid = lax.axis_index("s") * _NC + lax.axis_index("c")
    s0 = wid * _PCH
    gsems = (gs0, gs1, gs2, gs3)

    icps = [
        pltpu.async_copy(x_hbm.at[b, pl.ds(s0, _PCH)], idx_v.at[b], isem)
        for b in range(_B)
    ]
    pcp = pltpu.async_copy(pos_hbm.at[pl.ds(s0, _PCH)], pos_v, psem)
    for cp in icps:
        cp.wait()

    gcps = [
        pltpu.async_copy(table_hbm.at[idx_v.at[b]], rows_v.at[b], gsems[b])
        for b in range(_B)
    ]
    pcp.wait()

    wcps = []
    for b in range(_B):
        gcps[b].wait()

        def add_row(r, carry, _b=b):
            for j in range(_D // _L):
                sl = pl.ds(j * _L, _L)
                rows_v[_b, r, sl] = rows_v[_b, r, sl] + pos_v[r, sl]
            return carry

        lax.fori_loop(0, _PCH, add_row, 0)
        wcps.append(pltpu.async_copy(
            rows_v.at[b], out_hbm.at[b, pl.ds(s0, _PCH)], ws))
    for cp in wcps:
        cp.wait()


def kernel(x, table, pos_table):
    return _emb(x, table, pos_table)
